# baseline (speedup 1.0000x reference)
import jax
import jax.numpy as jnp
from jax.experimental import pallas as pl
from jax.experimental.pallas import tpu as pltpu

MB = 1024 * 1024
BF16 = jnp.bfloat16


def _conv1_kernel(xl_ref, w_ref, b_ref, o_ref):
    best = None
    for a in range(2):
        for b in range(2):
            acc = None
            for di in range(2):
                for dj in range(2):
                    g = (a * 2 + b) * 4 + di * 2 + dj
                    if dj == 0:
                        slab = xl_ref[0, di:di + 56, :, 0:16, :]
                    else:
                        slab = jnp.concatenate(
                            [xl_ref[0, di:di + 56, 1:4, 0:16, :],
                             xl_ref[0, di:di + 56, 0:1, 1:17, :]], axis=1)
                    p = slab.reshape(3584, 48)
                    d = jax.lax.dot_general(
                        p, w_ref[g], (((1,), (0,)), ((), ())),
                        preferred_element_type=jnp.float32)
                    acc = d if acc is None else acc + d
            best = acc if best is None else jnp.maximum(best, acc)
    z = jnp.maximum(best + b_ref[...], 0.0).astype(o_ref.dtype)
    o_ref[0, 0:14, :, :, 0:16] = z.reshape(14, 4, 4, 16, 64)
    o_ref[0, 0:14, :, :, 16:24] = jnp.zeros((14, 4, 4, 8, 64), o_ref.dtype)
    o_ref[0, 14:16] = jnp.zeros((2, 4, 4, 24, 64), o_ref.dtype)


def _conv2_kernel(zl_ref, w_ref, b_ref, o_ref):
    best = None
    for a in range(2):
        for b in range(2):
            acc = None
            for i in range(5):
                u, di = (2 * a + i) % 4, (2 * a + i) // 4
                for j in range(5):
                    v, dj = (2 * b + j) % 4, (2 * b + j) // 4
                    p = zl_ref[0, di:di + 14, u, v, dj:dj + 16, :]
                    d = jax.lax.dot_general(
                        p.reshape(224, 64), w_ref[i * 5 + j],
                        (((1,), (0,)), ((), ())),
                        preferred_element_type=jnp.float32)
                    acc = d if acc is None else acc + d
            best = acc if best is None else jnp.maximum(best, acc)
    z = jnp.maximum(best + b_ref[...], 0.0).astype(BF16)
    eye = (jax.lax.broadcasted_iota(jnp.int32, (128, 128), 0) ==
           jax.lax.broadcasted_iota(jnp.int32, (128, 128), 1)).astype(BF16)
    t = jax.lax.dot_general(eye, z, (((1,), (1,)), ((), ())),
                            preferred_element_type=jnp.float32)
    o_ref[0] = t.astype(o_ref.dtype)


def _fc_kernel(x_ref, w1_ref, b1_ref, w2_ref, b2_ref, o_ref):
    h = jax.lax.dot_general(
        x_ref[...], w1_ref[...], (((1,), (1,)), ((), ())),
        preferred_element_type=jnp.float32)
    h = jnp.maximum(h + b1_ref[...], 0.0)
    z = jax.lax.dot_general(
        h, w2_ref[...], (((1,), (1,)), ((), ())),
        preferred_element_type=jnp.float32)
    z = jnp.maximum(z + b2_ref[...], 0.0)
    m = jnp.max(z, axis=-1, keepdims=True)
    e = jnp.exp(z - m)
    s = jnp.sum(e, axis=-1, keepdims=True)
    o_ref[...] = (z - m - jnp.log(s)).astype(o_ref.dtype)


def _pack_conv1_w(cw):
    z = jnp.zeros((1, 64), jnp.float32)
    mats = []
    for a in range(2):
        for b in range(2):
            for di in range(2):
                for dj in range(2):
                    rows = []
                    for c in range(3):
                        for pi in range(4):
                            for pj in range(4):
                                i = 4 * di + pi - 2 * a
                                j = 4 * dj + pj - 2 * b
                                if 0 <= i < 5 and 0 <= j < 5:
                                    rows.append(cw[:, c, i, j][None, :])
                                else:
                                    rows.append(z)
                    mats.append(jnp.concatenate(rows, axis=0))
    return jnp.stack(mats).astype(BF16)


def kernel(conv1_w, conv1_b, conv2_w, conv2_b, fc1_w, fc1_b, fc2_w, fc2_b, x):
    N = x.shape[0]

    xl = x.reshape(N, 3, 56, 4, 14, 4, 4).transpose(0, 2, 5, 4, 1, 3, 6)
    xl = xl.reshape(N, 56, 4, 14, 48).astype(BF16)
    xl = jnp.pad(xl, ((0, 0), (0, 8), (0, 0), (0, 10), (0, 0)))

    w1g = _pack_conv1_w(conv1_w)
    b1 = conv1_b.reshape(1, 64)

    zl = pl.pallas_call(
        _conv1_kernel,
        out_shape=jax.ShapeDtypeStruct((N, 16, 4, 4, 24, 64), BF16),
        grid_spec=pltpu.PrefetchScalarGridSpec(
            num_scalar_prefetch=0,
            grid=(N,),
            in_specs=[
                pl.BlockSpec((1, 64, 4, 24, 48), lambda n: (n, 0, 0, 0, 0)),
                pl.BlockSpec((16, 48, 64), lambda n: (0, 0, 0)),
                pl.BlockSpec((1, 64), lambda n: (0, 0)),
            ],
            out_specs=pl.BlockSpec((1, 16, 4, 4, 24, 64),
                                   lambda n: (n, 0, 0, 0, 0, 0)),
        ),
        compiler_params=pltpu.CompilerParams(
            dimension_semantics=("parallel",),
            vmem_limit_bytes=48 * MB),
    )(xl, w1g, b1)

    w2t = jnp.transpose(conv2_w, (2, 3, 1, 0)).reshape(25, 64, 128)
    b2 = conv2_b.reshape(1, 128)

    y = pl.pallas_call(
        _conv2_kernel,
        out_shape=jax.ShapeDtypeStruct((N, 128, 224), BF16),
        grid_spec=pltpu.PrefetchScalarGridSpec(
            num_scalar_prefetch=0,
            grid=(N,),
            in_specs=[
                pl.BlockSpec((1, 16, 4, 4, 24, 64),
                             lambda n: (n, 0, 0, 0, 0, 0)),
                pl.BlockSpec((25, 64, 128), lambda n: (0, 0, 0)),
                pl.BlockSpec((1, 128), lambda n: (0, 0)),
            ],
            out_specs=pl.BlockSpec((1, 128, 224), lambda n: (n, 0, 0)),
        ),
        compiler_params=pltpu.CompilerParams(
            dimension_semantics=("parallel",),
            vmem_limit_bytes=48 * MB),
    )(zl, w2t.astype(BF16), b2)

    xf = y.reshape(N, 128, 14, 16)[:, :, :13, :13]
    xf = xf.reshape(N, 128 * 169).astype(jnp.float32)

    return pl.pallas_call(
        _fc_kernel,
        out_shape=jax.ShapeDtypeStruct((N, 2), jnp.float32),
        grid_spec=pltpu.PrefetchScalarGridSpec(
            num_scalar_prefetch=0,
            grid=(1,),
            in_specs=[
                pl.BlockSpec((N, 128 * 169), lambda i: (0, 0)),
                pl.BlockSpec((128, 128 * 169), lambda i: (0, 0)),
                pl.BlockSpec((1, 128), lambda i: (0, 0)),
                pl.BlockSpec((2, 128), lambda i: (0, 0)),
                pl.BlockSpec((1, 2), lambda i: (0, 0)),
            ],
            out_specs=pl.BlockSpec((N, 2), lambda i: (0, 0)),
        ),
        compiler_params=pltpu.CompilerParams(
            dimension_semantics=("arbitrary",),
            vmem_limit_bytes=48 * MB),
    )(xf, fc1_w, fc1_b.reshape(1, 128), fc2_w, fc2_b.reshape(1, 2))
